# baseline (speedup 1.0000x reference)
import functools

import jax
import jax.numpy as jnp
from jax import lax
from jax.experimental import pallas as pl
from jax.experimental.pallas import tpu as pltpu
from jax.experimental.pallas import tpu_sc as plsc

_NC = 2
_NS = 16
_NW = _NC * _NS
_CHUNK = 80
_NBUF = 4


def _sc_gather_pair(exe_idx, skill_idx, exe_table, skill_table):
    n = exe_idx.shape[0]
    d = exe_table.shape[1]
    per_w = n // _NW
    n_chunks = per_w // _CHUNK
    mesh = plsc.VectorSubcoreMesh(
        core_axis_name="c", subcore_axis_name="s",
        num_cores=_NC, num_subcores=_NS,
    )

    @functools.partial(
        pl.kernel,
        mesh=mesh,
        out_type=[
            jax.ShapeDtypeStruct((n, d), jnp.float32),
            jax.ShapeDtypeStruct((n, d), jnp.float32),
        ],
        scratch_types=[
            pltpu.VMEM((per_w,), jnp.int32),
            pltpu.VMEM((per_w,), jnp.int32),
            pltpu.VMEM((_NBUF, _CHUNK, d), jnp.float32),
            pltpu.VMEM((_NBUF, _CHUNK, d), jnp.float32),
            pltpu.SemaphoreType.DMA,
            pltpu.SemaphoreType.DMA,
        ],
    )
    def gather_kernel(exe_idx_hbm, skill_idx_hbm, exe_tab_hbm, skill_tab_hbm,
                      out_exe_hbm, out_skill_hbm,
                      idx_e, idx_s, rows_e, rows_s, sem_g, sem_w):
        wid = lax.axis_index("s") * _NC + lax.axis_index("c")
        base = wid * per_w
        pltpu.sync_copy(exe_idx_hbm.at[pl.ds(base, per_w)], idx_e)
        pltpu.sync_copy(skill_idx_hbm.at[pl.ds(base, per_w)], idx_s)
        n_groups = n_chunks // _NBUF

        def body(g, carry):
            gb = g * _NBUF * _CHUNK
            cps = []
            for k in range(_NBUF):
                io = gb + k * _CHUNK
                cps.append(pltpu.async_copy(
                    exe_tab_hbm.at[idx_e.at[pl.ds(io, _CHUNK)]],
                    rows_e.at[k], sem_g))
                cps.append(pltpu.async_copy(
                    skill_tab_hbm.at[idx_s.at[pl.ds(io, _CHUNK)]],
                    rows_s.at[k], sem_g))
            wcs = []
            for k in range(_NBUF):
                cps[2 * k].wait()
                cps[2 * k + 1].wait()
                off = base + gb + k * _CHUNK
                wcs.append(pltpu.async_copy(
                    rows_e.at[k], out_exe_hbm.at[pl.ds(off, _CHUNK)], sem_w))
                wcs.append(pltpu.async_copy(
                    rows_s.at[k], out_skill_hbm.at[pl.ds(off, _CHUNK)], sem_w))
            for w in wcs:
                w.wait()
            return carry

        lax.fori_loop(0, n_groups, body, 0)

    return gather_kernel(exe_idx, skill_idx, exe_table, skill_table)


def _tc_enc_body(seqs, bsz, comb, x, pos, W, b, enc_o):
    Wv = W[...]
    bv = b[...]
    d = bv.shape[1]
    dn = (((0,), (0,)), ((), ()))
    ones_row = jnp.ones((1, bsz), jnp.float32)
    ones128 = jnp.ones((1, d), jnp.float32)
    xw = jnp.dot(x[...], Wv, preferred_element_type=jnp.float32)
    for j in range(seqs):
        cj = comb[j]
        posr = pos[j]
        ids_row = (cj[0:1, :] + cj[1:2, :]).astype(jnp.float32)
        a_enc = jnp.concatenate([ids_row, ones_row], axis=0)
        b_enc = jnp.concatenate([ones128, bv + posr], axis=0)
        enc_o[pl.ds(j * bsz, bsz), :] = (
            xw[j * bsz:(j + 1) * bsz, :]
            + lax.dot_general(a_enc, b_enc, dn,
                              preferred_element_type=jnp.float32))


def _tc_dec_body(seqs, bsz, comb, pos, etW, etb, resp, dec_o):
    etWv = etW[...]
    etbv = etb[...]
    respv = resp[...]
    dn = (((0,), (0,)), ((), ()))
    ones_row = jnp.ones((1, bsz), jnp.float32)
    for j in range(seqs):
        cj = comb[j]
        posr = pos[j]
        r_row = cj[2:3, :]
        et_row = lax.bitcast_convert_type(cj[3:4, :], jnp.float32)
        oh = jnp.concatenate(
            [(r_row == t).astype(jnp.float32) for t in range(respv.shape[0])],
            axis=0,
        )
        a_dec = jnp.concatenate([oh, et_row, ones_row], axis=0)
        b_dec = jnp.concatenate([respv, etWv, etbv + posr], axis=0)
        dec_o[pl.ds(j * bsz, bsz), :] = lax.dot_general(
            a_dec, b_dec, dn, preferred_element_type=jnp.float32)


def _tc_out_body(y, gexe, gskill, W, b, out_o):
    out_o[...] = (
        jnp.dot(y[...], W[...], preferred_element_type=jnp.float32)
        + b[...] + gexe[...] + gskill[...]
    )


def _tc_enc(comb, x, pos, W, b, seqs, interpret=False):
    n, nlp = x.shape
    d = W.shape[1]
    s_dim = pos.shape[0]
    bsz = n // s_dim
    rows = seqs * bsz
    grid = (n // rows,)
    row_spec = lambda w: pl.BlockSpec((rows, w), lambda i: (i, 0))
    full = lambda h, w: pl.BlockSpec((h, w), lambda i: (0, 0))
    return pl.pallas_call(
        functools.partial(_tc_enc_body, seqs, bsz),
        grid=grid,
        in_specs=[
            pl.BlockSpec((seqs, 4, bsz), lambda i: (i, 0, 0)),
            row_spec(nlp),
            pl.BlockSpec((seqs, 1, d), lambda i: (i, 0, 0)),
            full(nlp, d), full(1, d),
        ],
        out_specs=[row_spec(d)],
        out_shape=[jax.ShapeDtypeStruct((n, d), jnp.float32)],
        compiler_params=pltpu.CompilerParams(
            dimension_semantics=("parallel",),
        ),
        interpret=interpret,
    )(comb, x, pos, W, b)[0]


def _tc_dec(comb, pos, etW, etb, resp, n, seqs, interpret=False):
    d = etW.shape[1]
    s_dim = pos.shape[0]
    bsz = n // s_dim
    rows = seqs * bsz
    grid = (n // rows,)
    full = lambda h, w: pl.BlockSpec((h, w), lambda i: (0, 0))
    return pl.pallas_call(
        functools.partial(_tc_dec_body, seqs, bsz),
        grid=grid,
        in_specs=[
            pl.BlockSpec((seqs, 4, bsz), lambda i: (i, 0, 0)),
            pl.BlockSpec((seqs, 1, d), lambda i: (i, 0, 0)),
            full(1, d), full(1, d), full(resp.shape[0], d),
        ],
        out_specs=[pl.BlockSpec((rows, d), lambda i: (i, 0))],
        out_shape=[jax.ShapeDtypeStruct((n, d), jnp.float32)],
        compiler_params=pltpu.CompilerParams(
            dimension_semantics=("parallel",),
        ),
        interpret=interpret,
    )(comb, pos, etW, etb, resp)[0]


def _tc_out(y, g_exe, g_skill, W, b, rows, interpret=False):
    n, nlp = y.shape
    d = W.shape[1]
    grid = (n // rows,)
    row_spec = lambda w: pl.BlockSpec((rows, w), lambda i: (i, 0))
    return pl.pallas_call(
        _tc_out_body,
        grid=grid,
        in_specs=[
            row_spec(nlp), row_spec(d), row_spec(d),
            pl.BlockSpec((nlp, d), lambda i: (0, 0)),
            pl.BlockSpec((1, d), lambda i: (0, 0)),
        ],
        out_specs=[row_spec(d)],
        out_shape=[jax.ShapeDtypeStruct((n, d), jnp.float32)],
        compiler_params=pltpu.CompilerParams(
            dimension_semantics=("parallel",),
        ),
        interpret=interpret,
    )(y, g_exe, g_skill, W, b)[0]


def kernel(input_nlp_embedding, input_exercise, input_skill, input_r,
           in_elapsed_time, output_nlp_embedding, out_exercise, out_skill,
           exercise_table, skill_table, response_table, pos_table,
           nlp_W, nlp_b, et_W, et_b):
    b_dim, s_dim, nlp = input_nlp_embedding.shape
    d = nlp_W.shape[1]
    n = b_dim * s_dim

    x_in = jnp.transpose(input_nlp_embedding, (1, 0, 2)).reshape(n, nlp)
    x_out = jnp.transpose(output_nlp_embedding, (1, 0, 2)).reshape(n, nlp)
    comb = jnp.concatenate([
        jnp.transpose(input_exercise).reshape(s_dim, 1, b_dim),
        jnp.transpose(input_skill).reshape(s_dim, 1, b_dim),
        jnp.transpose(input_r).reshape(s_dim, 1, b_dim),
        lax.bitcast_convert_type(
            jnp.transpose(in_elapsed_time, (1, 2, 0)), jnp.int32),
    ], axis=1)
    oexe_t = jnp.transpose(out_exercise).reshape(n)
    oskill_t = jnp.transpose(out_skill).reshape(n)

    g_exe, g_skill = _sc_gather_pair(
        oexe_t, oskill_t, exercise_table, skill_table,
    )

    pos3 = pos_table.reshape(s_dim, 1, d)
    dec2 = _tc_dec(comb, pos3, et_W, et_b.reshape(1, d), response_table,
                   n, seqs=2)
    enc2 = _tc_enc(comb, x_in, pos3, nlp_W, nlp_b.reshape(1, d), seqs=2)
    out2 = _tc_out(x_out, g_exe, g_skill, nlp_W, nlp_b.reshape(1, d),
                   rows=2 * b_dim)

    def back(a2):
        return jnp.transpose(a2.reshape(s_dim, b_dim, d), (1, 0, 2))

    return (back(enc2), back(dec2), back(out2))


# speedup vs baseline: 1.1517x; 1.1517x over previous
import functools

import jax
import jax.numpy as jnp
from jax import lax
from jax.experimental import pallas as pl
from jax.experimental.pallas import tpu as pltpu
from jax.experimental.pallas import tpu_sc as plsc

_NC = 2
_NS = 16
_NW = _NC * _NS
_CHUNK = 80
_NBUF = 4


def _sc_gather_sum(exe_idx, skill_idx, exe_table, skill_table):
    n = exe_idx.shape[0]
    d = exe_table.shape[1]
    per_w = n // _NW
    n_chunks = per_w // _CHUNK
    mesh = plsc.VectorSubcoreMesh(
        core_axis_name="c", subcore_axis_name="s",
        num_cores=_NC, num_subcores=_NS,
    )

    @functools.partial(
        pl.kernel,
        mesh=mesh,
        out_type=jax.ShapeDtypeStruct((n, d), jnp.float32),
        scratch_types=[
            pltpu.VMEM((per_w,), jnp.int32),
            pltpu.VMEM((per_w,), jnp.int32),
            pltpu.VMEM((_NBUF, _CHUNK, d), jnp.float32),
            pltpu.SemaphoreType.DMA,
            pltpu.SemaphoreType.DMA,
            pltpu.SemaphoreType.DMA,
        ],
    )
    def gather_kernel(exe_idx_hbm, skill_idx_hbm, exe_tab_hbm, skill_tab_hbm,
                      out_hbm, idx_e, idx_s, rows, sem_e, sem_s, sem_w):
        wid = lax.axis_index("s") * _NC + lax.axis_index("c")
        base = wid * per_w
        pltpu.sync_copy(exe_idx_hbm.at[pl.ds(base, per_w)], idx_e)
        pltpu.sync_copy(skill_idx_hbm.at[pl.ds(base, per_w)], idx_s)
        n_groups = n_chunks // _NBUF

        def body(g, carry):
            gb = g * _NBUF * _CHUNK
            cps = []
            for k in range(_NBUF):
                io = gb + k * _CHUNK
                cps.append(pltpu.async_copy(
                    exe_tab_hbm.at[idx_e.at[pl.ds(io, _CHUNK)]],
                    rows.at[k], sem_e))
            scs = []
            for k in range(_NBUF):
                io = gb + k * _CHUNK
                cps[k].wait()
                scs.append(pltpu.async_copy(
                    skill_tab_hbm.at[idx_s.at[pl.ds(io, _CHUNK)]],
                    rows.at[k], sem_s, add=True))
            wcs = []
            for k in range(_NBUF):
                scs[k].wait()
                off = base + gb + k * _CHUNK
                wcs.append(pltpu.async_copy(
                    rows.at[k], out_hbm.at[pl.ds(off, _CHUNK)], sem_w))
            for w in wcs:
                w.wait()
            return carry

        lax.fori_loop(0, n_groups, body, 0)

    return gather_kernel(exe_idx, skill_idx, exe_table, skill_table)


def _tc_enc_body(seqs, bsz, comb, x, pos, W, b, enc_o):
    Wv = W[...]
    bv = b[...]
    d = bv.shape[1]
    dn = (((0,), (0,)), ((), ()))
    ones_row = jnp.ones((1, bsz), jnp.float32)
    ones128 = jnp.ones((1, d), jnp.float32)
    xw = jnp.dot(x[...], Wv, preferred_element_type=jnp.float32)
    for j in range(seqs):
        cj = comb[j]
        posr = pos[j]
        ids_row = (cj[0:1, :] + cj[1:2, :]).astype(jnp.float32)
        a_enc = jnp.concatenate([ids_row, ones_row], axis=0)
        b_enc = jnp.concatenate([ones128, bv + posr], axis=0)
        enc_o[pl.ds(j * bsz, bsz), :] = (
            xw[j * bsz:(j + 1) * bsz, :]
            + lax.dot_general(a_enc, b_enc, dn,
                              preferred_element_type=jnp.float32))


def _tc_dec_body(seqs, bsz, comb, pos, etW, etb, resp, dec_o):
    etWv = etW[...]
    etbv = etb[...]
    respv = resp[...]
    dn = (((0,), (0,)), ((), ()))
    ones_row = jnp.ones((1, bsz), jnp.float32)
    for j in range(seqs):
        cj = comb[j]
        posr = pos[j]
        r_row = cj[2:3, :]
        et_row = lax.bitcast_convert_type(cj[3:4, :], jnp.float32)
        oh = jnp.concatenate(
            [(r_row == t).astype(jnp.float32) for t in range(respv.shape[0])],
            axis=0,
        )
        a_dec = jnp.concatenate([oh, et_row, ones_row], axis=0)
        b_dec = jnp.concatenate([respv, etWv, etbv + posr], axis=0)
        dec_o[pl.ds(j * bsz, bsz), :] = lax.dot_general(
            a_dec, b_dec, dn, preferred_element_type=jnp.float32)


def _tc_out_body(y, gsum, W, b, out_o):
    out_o[...] = (
        jnp.dot(y[...], W[...], preferred_element_type=jnp.float32)
        + b[...] + gsum[...]
    )


def _tc_enc(comb, x, pos, W, b, seqs, interpret=False):
    n, nlp = x.shape
    d = W.shape[1]
    s_dim = pos.shape[0]
    bsz = n // s_dim
    rows = seqs * bsz
    grid = (n // rows,)
    row_spec = lambda w: pl.BlockSpec((rows, w), lambda i: (i, 0))
    full = lambda h, w: pl.BlockSpec((h, w), lambda i: (0, 0))
    return pl.pallas_call(
        functools.partial(_tc_enc_body, seqs, bsz),
        grid=grid,
        in_specs=[
            pl.BlockSpec((seqs, 4, bsz), lambda i: (i, 0, 0)),
            row_spec(nlp),
            pl.BlockSpec((seqs, 1, d), lambda i: (i, 0, 0)),
            full(nlp, d), full(1, d),
        ],
        out_specs=[row_spec(d)],
        out_shape=[jax.ShapeDtypeStruct((n, d), jnp.float32)],
        compiler_params=pltpu.CompilerParams(
            dimension_semantics=("parallel",),
        ),
        interpret=interpret,
    )(comb, x, pos, W, b)[0]


def _tc_dec(comb, pos, etW, etb, resp, n, seqs, interpret=False):
    d = etW.shape[1]
    s_dim = pos.shape[0]
    bsz = n // s_dim
    rows = seqs * bsz
    grid = (n // rows,)
    full = lambda h, w: pl.BlockSpec((h, w), lambda i: (0, 0))
    return pl.pallas_call(
        functools.partial(_tc_dec_body, seqs, bsz),
        grid=grid,
        in_specs=[
            pl.BlockSpec((seqs, 4, bsz), lambda i: (i, 0, 0)),
            pl.BlockSpec((seqs, 1, d), lambda i: (i, 0, 0)),
            full(1, d), full(1, d), full(resp.shape[0], d),
        ],
        out_specs=[pl.BlockSpec((rows, d), lambda i: (i, 0))],
        out_shape=[jax.ShapeDtypeStruct((n, d), jnp.float32)],
        compiler_params=pltpu.CompilerParams(
            dimension_semantics=("parallel",),
        ),
        interpret=interpret,
    )(comb, pos, etW, etb, resp)[0]


def _tc_out(y, g_sum, W, b, rows, interpret=False):
    n, nlp = y.shape
    d = W.shape[1]
    grid = (n // rows,)
    row_spec = lambda w: pl.BlockSpec((rows, w), lambda i: (i, 0))
    return pl.pallas_call(
        _tc_out_body,
        grid=grid,
        in_specs=[
            row_spec(nlp), row_spec(d),
            pl.BlockSpec((nlp, d), lambda i: (0, 0)),
            pl.BlockSpec((1, d), lambda i: (0, 0)),
        ],
        out_specs=[row_spec(d)],
        out_shape=[jax.ShapeDtypeStruct((n, d), jnp.float32)],
        compiler_params=pltpu.CompilerParams(
            dimension_semantics=("parallel",),
        ),
        interpret=interpret,
    )(y, g_sum, W, b)[0]


def kernel(input_nlp_embedding, input_exercise, input_skill, input_r,
           in_elapsed_time, output_nlp_embedding, out_exercise, out_skill,
           exercise_table, skill_table, response_table, pos_table,
           nlp_W, nlp_b, et_W, et_b):
    b_dim, s_dim, nlp = input_nlp_embedding.shape
    d = nlp_W.shape[1]
    n = b_dim * s_dim

    x_in = jnp.transpose(input_nlp_embedding, (1, 0, 2)).reshape(n, nlp)
    x_out = jnp.transpose(output_nlp_embedding, (1, 0, 2)).reshape(n, nlp)
    comb = jnp.concatenate([
        jnp.transpose(input_exercise).reshape(s_dim, 1, b_dim),
        jnp.transpose(input_skill).reshape(s_dim, 1, b_dim),
        jnp.transpose(input_r).reshape(s_dim, 1, b_dim),
        lax.bitcast_convert_type(
            jnp.transpose(in_elapsed_time, (1, 2, 0)), jnp.int32),
    ], axis=1)
    oexe_t = jnp.transpose(out_exercise).reshape(n)
    oskill_t = jnp.transpose(out_skill).reshape(n)

    g_sum = _sc_gather_sum(oexe_t, oskill_t, exercise_table, skill_table)

    pos3 = pos_table.reshape(s_dim, 1, d)
    dec2 = _tc_dec(comb, pos3, et_W, et_b.reshape(1, d), response_table,
                   n, seqs=10)
    enc2 = _tc_enc(comb, x_in, pos3, nlp_W, nlp_b.reshape(1, d), seqs=2)
    out2 = _tc_out(x_out, g_sum, nlp_W, nlp_b.reshape(1, d),
                   rows=2 * b_dim)

    def back(a2):
        return jnp.transpose(a2.reshape(s_dim, b_dim, d), (1, 0, 2))

    return (back(enc2), back(dec2), back(out2))
